# baseline (speedup 1.0000x reference)
import jax
import jax.numpy as jnp
from jax.experimental import pallas as pl

_NN = 100000
_EE = 1600000
_NBLK = 10000
_EBLK = 8000


def _row_call(body, out_dim, block, *args):
    rows = args[0].shape[0]
    grid = rows // block
    in_specs = []
    for a in args:
        nd = a.ndim
        if a.shape[0] == rows:
            in_specs.append(
                pl.BlockSpec((block,) + a.shape[1:],
                             lambda i, _nd=nd: (i,) + (0,) * (_nd - 1)))
        else:
            in_specs.append(
                pl.BlockSpec(a.shape, lambda i, _nd=nd: (0,) * _nd))
    return pl.pallas_call(
        body,
        grid=(grid,),
        in_specs=in_specs,
        out_specs=pl.BlockSpec((block, out_dim), lambda i: (i, 0)),
        out_shape=jax.ShapeDtypeStruct((rows, out_dim), jnp.float32),
    )(*args)


def _mm_body(a_ref, w_ref, b_ref, o_ref):
    o_ref[...] = jnp.dot(a_ref[...], w_ref[...],
                         preferred_element_type=jnp.float32) + b_ref[...]


def _mm2_body(a_ref, wl_ref, wr_ref, o_ref):
    a = a_ref[...]
    l = jnp.dot(a, wl_ref[...], preferred_element_type=jnp.float32)
    r = jnp.dot(a, wr_ref[...], preferred_element_type=jnp.float32)
    o_ref[...] = jnp.concatenate([l, r], axis=-1)


def _bnmm_body(z_ref, g_ref, be_ref, mu_ref, var_ref, w_ref, b_ref, o_ref):
    h = g_ref[...] * (z_ref[...] - mu_ref[...]) * jax.lax.rsqrt(
        var_ref[...] + 1e-5) + be_ref[...]
    h = jnp.maximum(h, 0.0)
    o_ref[...] = jnp.dot(h, w_ref[...],
                         preferred_element_type=jnp.float32) + b_ref[...]


def _edge_body(xs_ref, xd_ref, att_ref, o_ref):
    s = xs_ref[...] + xd_ref[...]
    l = jnp.where(s > 0, s, 0.2 * s)
    o_ref[...] = jnp.sum(att_ref[...] * l, axis=-1, keepdims=True)


def _exp_body(e_ref, m_ref, o_ref):
    o_ref[...] = jnp.exp(e_ref[...] - m_ref[...])


def _msg_body(ex_ref, den_ref, xs_ref, o_ref):
    o_ref[...] = ex_ref[...] / (den_ref[...] + 1e-16) * xs_ref[...]


def _prodmm_body(a_ref, b_ref, w_ref, bias_ref, o_ref):
    o_ref[...] = jnp.dot(a_ref[...] * b_ref[...], w_ref[...],
                         preferred_element_type=jnp.float32) + bias_ref[...]


def _pair_softmax_body(z_ref, o_ref):
    zz = z_ref[...]
    m = jnp.max(zz, axis=-1, keepdims=True)
    ex = jnp.exp(zz - m)
    o_ref[...] = ex / jnp.sum(ex, axis=-1, keepdims=True)


def _mlp_stack(h, p, pre, block):
    z0 = _row_call(_mm_body, p[pre + '_W0'].shape[1], block,
                   h, p[pre + '_W0'], p[pre + '_b0'].reshape(1, -1))
    mu0 = jnp.mean(z0, axis=0, keepdims=True)
    var0 = jnp.var(z0, axis=0, keepdims=True)
    z1 = _row_call(_bnmm_body, p[pre + '_W1'].shape[1], block,
                   z0, p[pre + '_g0'].reshape(1, -1),
                   p[pre + '_be0'].reshape(1, -1), mu0, var0,
                   p[pre + '_W1'], p[pre + '_b1'].reshape(1, -1))
    mu1 = jnp.mean(z1, axis=0, keepdims=True)
    var1 = jnp.var(z1, axis=0, keepdims=True)
    return _row_call(_bnmm_body, p[pre + '_W2'].shape[1], block,
                     z1, p[pre + '_g1'].reshape(1, -1),
                     p[pre + '_be1'].reshape(1, -1), mu1, var1,
                     p[pre + '_W2'], p[pre + '_b2'].reshape(1, -1))


def _gat(h, ei, wl, wr, att, bias):
    src, dst = ei[0], ei[1]
    xlr = _row_call(_mm2_body, 8, _NBLK, h, wl, wr)
    xl, xr = xlr[:, :4], xlr[:, 4:]
    xs = xl[src]
    xd = xr[dst]
    e = _row_call(_edge_body, 1, _EBLK, xs, xd, att.reshape(1, -1))
    e1 = e[:, 0]
    emax = jax.ops.segment_max(e1, dst, num_segments=_NN)
    emax = jnp.where(jnp.isfinite(emax), emax, 0.0)
    ex = _row_call(_exp_body, 1, _EBLK, e, emax[dst][:, None])
    denom = jax.ops.segment_sum(ex[:, 0], dst, num_segments=_NN)
    msg = _row_call(_msg_body, 4, _EBLK, ex, denom[dst][:, None], xs)
    out = jax.ops.segment_sum(msg, dst, num_segments=_NN)
    return out + bias


def kernel(x, ei_pre, ei_rev, ei_con, params):
    p = params
    h = _mlp_stack(x, p, 'enc', _NBLK)
    rel = [('pre', ei_pre), ('rev', ei_rev), ('con', ei_con)]
    layer_outs = []
    for l in range(3):
        outs = []
        for name, ei in rel:
            outs.append(_gat(h, ei,
                             p['Wl_%d_%s' % (l, name)],
                             p['Wr_%d_%s' % (l, name)],
                             p['att_%d_%s' % (l, name)],
                             p['bias_%d_%s' % (l, name)]))
        h = jax.nn.relu(jnp.concatenate(outs, axis=-1))
        layer_outs.append(h)
    cat = jnp.concatenate(layer_outs, axis=-1)
    a = cat[ei_con[0]]
    b = cat[ei_con[1]]
    z0 = _row_call(_prodmm_body, 4, _EBLK,
                   a, b, p['dec_W0'], p['dec_b0'].reshape(1, -1))
    mu0 = jnp.mean(z0, axis=0, keepdims=True)
    var0 = jnp.var(z0, axis=0, keepdims=True)
    z1 = _row_call(_bnmm_body, 4, _EBLK,
                   z0, p['dec_g0'].reshape(1, -1),
                   p['dec_be0'].reshape(1, -1), mu0, var0,
                   p['dec_W1'], p['dec_b1'].reshape(1, -1))
    mu1 = jnp.mean(z1, axis=0, keepdims=True)
    var1 = jnp.var(z1, axis=0, keepdims=True)
    z2 = _row_call(_bnmm_body, 1, _EBLK,
                   z1, p['dec_g1'].reshape(1, -1),
                   p['dec_be1'].reshape(1, -1), mu1, var1,
                   p['dec_W2'], p['dec_b2'].reshape(1, -1))
    zz = z2.reshape(_EE // 2, 2)
    sm = _row_call(_pair_softmax_body, 2, _EBLK, zz)
    return sm.reshape(_EE, 1)
